# baseline (speedup 1.0000x reference)
import functools

import jax
import jax.numpy as jnp
from jax import lax
from jax.experimental import pallas as pl
from jax.experimental.pallas import tpu as pltpu
from jax.experimental.pallas import tpu_sc as plsc

D = 768
B_TOTAL = 4 * 8192
NC = 2
NS = 16
NW = NC * NS
B_PER_W = B_TOTAL // NW
CHUNK = 32
NCHUNK = B_PER_W // CHUNK
NSLOT = 1
HALF = CHUNK // 2
DIRECT_SET = (0, 11, 22)


def _posenc_body(pe_hbm, idx_hbm, out_hbm, idx_v, rows0, rows1, shared,
                 gs0, gs1, psem, ds0, ds1):
    sid = lax.axis_index("s")
    wid = sid * NC + lax.axis_index("c")
    base = wid * B_PER_W
    pltpu.sync_copy(idx_hbm.at[wid], idx_v)

    rows = (rows0, rows1)
    gsem = (gs0, gs1)

    def is_direct(c):
        return c in DIRECT_SET

    dsem = (ds0, ds1)

    pending = []

    def wait(tag):
        for i, (t, op) in enumerate(pending):
            if t == tag:
                op.wait()
                del pending[i]
                return

    gops = [None] * (NCHUNK + 1)
    gops[0] = pltpu.async_copy(pe_hbm.at[idx_v.at[0]], rows[0], gsem[0])
    for c in range(NCHUNK):
        b = c % 2
        if c + 1 < NCHUNK:
            nb = (c + 1) % 2
            if c >= 1:
                wait(("out", c - 1))
            gops[c + 1] = pltpu.async_copy(pe_hbm.at[idx_v.at[c + 1]],
                                           rows[nb], gsem[nb])
        gops[c].wait()
        if is_direct(c):
            dst = out_hbm.at[pl.ds(base + c * CHUNK, CHUNK)]
            pending.append((("out", c), pltpu.async_copy(rows[b], dst, gsem[b])))
        else:
            for h in range(2):
                wait(("drain", h))
                push = pltpu.async_copy(rows[b].at[pl.ds(h * HALF, HALF)],
                                        shared.at[sid, h], psem)
                push.wait()
                hdst = out_hbm.at[pl.ds(base + c * CHUNK + h * HALF, HALF)]
                pending.append((("drain", h),
                                pltpu.async_copy(shared.at[sid, h], hdst, dsem[h])))

    for _, op in pending:
        op.wait()


@jax.jit
def _posenc(pe, idx):
    k = pl.kernel(
        _posenc_body,
        out_type=jax.ShapeDtypeStruct((B_TOTAL, D), jnp.float32),
        mesh=plsc.VectorSubcoreMesh(core_axis_name="c", subcore_axis_name="s"),
        scratch_types=[
            pltpu.VMEM((NCHUNK, CHUNK), jnp.int32),
            pltpu.VMEM((CHUNK, D), jnp.float32),
            pltpu.VMEM((CHUNK, D), jnp.float32),
            pltpu.VMEM_SHARED((NS, 2, HALF, D), jnp.float32),
            pltpu.SemaphoreType.DMA,
            pltpu.SemaphoreType.DMA,
            pltpu.SemaphoreType.DMA,
            pltpu.SemaphoreType.DMA,
            pltpu.SemaphoreType.DMA,
        ],
    )
    return k(pe, idx)


def kernel(x, pe):
    idx = x.astype(jnp.int32).reshape(NW, NCHUNK, CHUNK)
    out = _posenc(pe, idx)
    return out.reshape(x.shape[0], x.shape[1], D)


# speedup vs baseline: 1.0053x; 1.0053x over previous
import functools

import jax
import jax.numpy as jnp
from jax import lax
from jax.experimental import pallas as pl
from jax.experimental.pallas import tpu as pltpu
from jax.experimental.pallas import tpu_sc as plsc

D = 768
B = 4
L = 8192
NC = 2
NS = 16
NW = NC * NS
B_PER_W = (B * L) // NW
W_PER_ROW = L // B_PER_W
CHUNK = 64
NCHUNK = B_PER_W // CHUNK


def _posenc_body(x_hbm, pe_hbm, out_hbm, idx_v, rows0, rows1, gs0, gs1, ss0, ss1):
    wid = lax.axis_index("s") * NC + lax.axis_index("c")
    row = wid // W_PER_ROW
    col = (wid % W_PER_ROW) * B_PER_W
    pltpu.sync_copy(x_hbm.at[row, pl.ds(col, B_PER_W)], idx_v)

    rows = (rows0, rows1)
    gsem = (gs0, gs1)
    ssem = (ss0, ss1)

    gops = [None] * NCHUNK
    sops = [None] * NCHUNK
    gops[0] = pltpu.async_copy(pe_hbm.at[idx_v.at[pl.ds(0, CHUNK)]], rows[0], gs0)
    for c in range(NCHUNK):
        b = c % 2
        if c + 1 < NCHUNK:
            nb = (c + 1) % 2
            if c >= 1:
                sops[c - 1].wait()
            gops[c + 1] = pltpu.async_copy(
                pe_hbm.at[idx_v.at[pl.ds((c + 1) * CHUNK, CHUNK)]], rows[nb], gsem[nb])
        gops[c].wait()
        sops[c] = pltpu.async_copy(
            rows[b], out_hbm.at[row, pl.ds(col + c * CHUNK, CHUNK)], ssem[b])
    sops[NCHUNK - 2].wait()
    sops[NCHUNK - 1].wait()


@jax.jit
def _posenc(x, pe):
    k = pl.kernel(
        _posenc_body,
        out_type=jax.ShapeDtypeStruct((B, L, D), jnp.float32),
        mesh=plsc.VectorSubcoreMesh(core_axis_name="c", subcore_axis_name="s"),
        scratch_types=[
            pltpu.VMEM((B_PER_W,), jnp.int32),
            pltpu.VMEM((CHUNK, D), jnp.float32),
            pltpu.VMEM((CHUNK, D), jnp.float32),
            pltpu.SemaphoreType.DMA,
            pltpu.SemaphoreType.DMA,
            pltpu.SemaphoreType.DMA,
            pltpu.SemaphoreType.DMA,
        ],
    )
    return k(x, pe)


def kernel(x, pe):
    return _posenc(x.astype(jnp.int32), pe)
